# baseline (speedup 1.0000x reference)
import functools

import jax
import jax.numpy as jnp
from jax import lax
from jax.experimental import pallas as pl
from jax.experimental.pallas import tpu as pltpu
from jax.experimental.pallas import tpu_sc as plsc

N = 10000
E = 320000
D = 128
H = 64
C = 40
K = 10
ALPHA = 0.1

W = 48
NPAD = 10112
DUMMY = N
NC, NS = 2, 16
NWORK = NC * NS
CH = 128
NCHUNK = 80
EPT = CH * NCHUNK
EPAD = EPT * NWORK
RPT = NPAD // NS

SR = 4
SCH = SR * CH
NSUP = EPT // SCH

_sc_mesh = plsc.VectorSubcoreMesh(
    core_axis_name="c", subcore_axis_name="s", num_cores=NC, num_subcores=NS
)


def _deg_body(cidx_hbm, ones_hbm, z16_hbm, out_hbm, ones_v, cslab_v, acc_sh, sem):
    cid = lax.axis_index("c")
    sid = lax.axis_index("s")
    wid = cid * NS + sid

    pltpu.sync_copy(ones_hbm, ones_v)
    pltpu.sync_copy(
        z16_hbm.at[pl.ds(sid * RPT, RPT)], acc_sh.at[pl.ds(sid * RPT, RPT)]
    )
    pltpu.sync_copy(cidx_hbm.at[wid], cslab_v)
    plsc.subcore_barrier()

    @pl.loop(0, NCHUNK // 8, step=1)
    def _(j):
        pltpu.async_copy(ones_v, acc_sh.at[cslab_v.at[j]], sem, add=True)

    @pl.loop(0, NCHUNK // 8, step=1)
    def _(j):
        pltpu.make_async_copy(ones_v, acc_sh.at[cslab_v.at[0]], sem).wait()

    plsc.subcore_barrier()
    pltpu.sync_copy(
        acc_sh.at[pl.ds(sid * RPT, RPT)], out_hbm.at[cid, pl.ds(sid * RPT, RPT)]
    )


@functools.partial(
    pl.kernel,
    out_type=jax.ShapeDtypeStruct((NC, NPAD, 16), jnp.float32),
    mesh=_sc_mesh,
    compiler_params=pltpu.CompilerParams(use_tc_tiling_on_sc=False),
    scratch_types=[
        pltpu.VMEM((8 * CH, 16), jnp.float32),
        pltpu.VMEM((NCHUNK // 8, 8 * CH), jnp.int32),
        pltpu.VMEM_SHARED((NPAD, 16), jnp.float32),
        pltpu.SemaphoreType.DMA,
    ],
)
def _deg_sc(cidx_hbm, ones_hbm, z16_hbm, out_hbm, ones_v, cslab_v, acc_sh, sem):
    _deg_body(cidx_hbm, ones_hbm, z16_hbm, out_hbm, ones_v, cslab_v, acc_sh, sem)


def _scat_body(y_hbm, ridx_hbm, cidx_hbm, z_hbm, out_hbm,
               rslab_v, cslab_v, bufs, acc_sh, gsems, ssems):
    cid = lax.axis_index("c")
    sid = lax.axis_index("s")
    wid = cid * NS + sid

    def fire_g(i, k):
        pltpu.async_copy(y_hbm.at[rslab_v.at[i]], bufs[k], gsems[k])

    def drain_g(k):
        pltpu.make_async_copy(y_hbm.at[rslab_v.at[0]], bufs[k], gsems[k]).wait()

    def fire_s(i, k):
        pltpu.async_copy(bufs[k], acc_sh.at[cslab_v.at[i]], ssems[k], add=True)

    def drain_s(k):
        pltpu.make_async_copy(bufs[k], acc_sh.at[cslab_v.at[0]], ssems[k]).wait()

    pltpu.sync_copy(
        z_hbm.at[pl.ds(sid * RPT, RPT)], acc_sh.at[pl.ds(sid * RPT, RPT)]
    )
    pltpu.sync_copy(ridx_hbm.at[wid], rslab_v)
    pltpu.sync_copy(cidx_hbm.at[wid], cslab_v)
    plsc.subcore_barrier()

    fire_g(0, 0)

    @pl.loop(0, NSUP, step=2)
    def _(i):
        drain_g(0)
        fire_s(i, 0)
        fire_g(i + 1, 1)
        drain_s(0)
        drain_g(1)
        fire_s(i + 1, 1)

        @pl.when(i + 2 < NSUP)
        def _():
            fire_g(i + 2, 0)

        drain_s(1)

    plsc.subcore_barrier()
    pltpu.sync_copy(
        acc_sh.at[pl.ds(sid * RPT, RPT)], out_hbm.at[cid, pl.ds(sid * RPT, RPT)]
    )


@functools.partial(
    pl.kernel,
    out_type=jax.ShapeDtypeStruct((NC, NPAD, W), jnp.float32),
    mesh=_sc_mesh,
    compiler_params=pltpu.CompilerParams(use_tc_tiling_on_sc=False),
    scratch_types=(
        [pltpu.VMEM((NSUP, SCH), jnp.int32)] * 2
        + [pltpu.VMEM((SCH, W), jnp.float32)] * 2
        + [pltpu.VMEM_SHARED((NPAD, W), jnp.float32)]
        + [pltpu.SemaphoreType.DMA] * 4
    ),
)
def _scat_sc(y_hbm, ridx_hbm, cidx_hbm, z_hbm, out_hbm,
             rslab_v, cslab_v, buf0, buf1, acc_sh, g0, g1, s0, s1):
    _scat_body(y_hbm, ridx_hbm, cidx_hbm, z_hbm, out_hbm,
               rslab_v, cslab_v, (buf0, buf1), acc_sh, (g0, g1), (s0, s1))


def _mlp_body(xp_ref, w0_ref, b0_ref, w1_ref, b1_ref, h_ref):
    g = jnp.dot(xp_ref[...], w0_ref[...], preferred_element_type=jnp.float32)
    g = jnp.maximum(g + b0_ref[...], 0.0)
    h_ref[...] = (
        jnp.dot(g, w1_ref[...], preferred_element_type=jnp.float32) + b1_ref[...]
    )


def _mlp_tc(xp, w0, b0r, w1p, b1p):
    return pl.pallas_call(
        _mlp_body,
        out_shape=jax.ShapeDtypeStruct((NPAD, W), jnp.float32),
    )(xp, w0, b0r, w1p, b1p)


def _prep_body(h_ref, degp_ref, dinv_ref, y0_ref):
    deg = degp_ref[0, :, 0:1] + degp_ref[1, :, 0:1] + 1.0
    dinv = jnp.broadcast_to(lax.rsqrt(deg), (NPAD, W))
    dinv_ref[...] = dinv
    y0_ref[...] = dinv * h_ref[...]


def _prep_tc(h, degp):
    return pl.pallas_call(
        _prep_body,
        out_shape=(
            jax.ShapeDtypeStruct((NPAD, W), jnp.float32),
            jax.ShapeDtypeStruct((NPAD, W), jnp.float32),
        ),
    )(h, degp)


def _combine_body(sp_ref, x_ref, h_ref, dinv_ref, xn_ref, yn_ref):
    dinv = dinv_ref[...]
    s = sp_ref[0] + sp_ref[1]
    xn = (1.0 - ALPHA) * (dinv * s + dinv * dinv * x_ref[...]) + ALPHA * h_ref[...]
    xn_ref[...] = xn
    yn_ref[...] = dinv * xn


def _combine_tc(sp, x, h, dinv):
    return pl.pallas_call(
        _combine_body,
        out_shape=(
            jax.ShapeDtypeStruct((NPAD, W), jnp.float32),
            jax.ShapeDtypeStruct((NPAD, W), jnp.float32),
        ),
    )(sp, x, h, dinv)


def _last_body(sp_ref, x_ref, h_ref, dinv_ref, out_ref):
    dinv = dinv_ref[...]
    s = sp_ref[0] + sp_ref[1]
    xn = (1.0 - ALPHA) * (dinv * s + dinv * dinv * x_ref[...]) + ALPHA * h_ref[...]
    mask = lax.broadcasted_iota(jnp.int32, (NPAD, W), 1) < C
    xm = jnp.where(mask, xn, -jnp.inf)
    m = jnp.max(xm, axis=1, keepdims=True)
    e = jnp.where(mask, jnp.exp(xm - m), 0.0)
    lse = jnp.log(jnp.sum(e, axis=1, keepdims=True)) + m
    out_ref[...] = xn - lse


def _last_tc(sp, x, h, dinv):
    return pl.pallas_call(
        _last_body,
        out_shape=jax.ShapeDtypeStruct((NPAD, W), jnp.float32),
    )(sp, x, h, dinv)


def kernel(inputs, edge_index, W0, b0, W1, b1):
    row = edge_index[0].astype(jnp.int32)
    col = edge_index[1].astype(jnp.int32)
    npad_e = EPAD - E
    ridx = jnp.concatenate([row, jnp.zeros((npad_e,), jnp.int32)])
    cidx = jnp.concatenate([col, jnp.full((npad_e,), DUMMY, jnp.int32)])
    ridx = ridx.reshape(NWORK, NSUP, SCH)
    cidx = cidx.reshape(NWORK, NSUP, SCH)
    cidx8 = cidx.reshape(NWORK, NCHUNK // 8, 8 * CH)

    xp = jnp.zeros((NPAD, D), jnp.float32).at[:N].set(inputs)
    w1p = jnp.zeros((H, W), jnp.float32).at[:, :C].set(W1)
    b1p = jnp.zeros((1, W), jnp.float32).at[0, :C].set(b1)
    b0r = b0.reshape(1, H)
    ones16 = jnp.ones((8 * CH, 16), jnp.float32)
    z16 = jnp.zeros((NPAD, 16), jnp.float32)
    z48 = jnp.zeros((NPAD, W), jnp.float32)

    degp = _deg_sc(cidx8, ones16, z16)
    h = _mlp_tc(xp, W0, b0r, w1p, b1p)
    dinv, y = _prep_tc(h, degp)
    x = h
    for _ in range(K - 1):
        sp = _scat_sc(y, ridx, cidx, z48)
        x, y = _combine_tc(sp, x, h, dinv)
    sp = _scat_sc(y, ridx, cidx, z48)
    out = _last_tc(sp, x, h, dinv)
    return out[:N, :C]


# speedup vs baseline: 2.3860x; 2.3860x over previous
import functools

import jax
import jax.numpy as jnp
from jax import lax
from jax.experimental import pallas as pl
from jax.experimental.pallas import tpu as pltpu
from jax.experimental.pallas import tpu_sc as plsc

N = 10000
E = 320000
D = 128
H = 64
C = 40
K = 10
ALPHA = 0.1

W = 48
NPAD = 10112
DUMMY = N
NC, NS = 2, 16
NWORK = NC * NS
CH = 128
NCHUNK = 80
EPT = CH * NCHUNK
EPAD = EPT * NWORK
RPT = NPAD // NS

SR = 4
SCH = SR * CH
NSUP = EPT // SCH

_sc_mesh = plsc.VectorSubcoreMesh(
    core_axis_name="c", subcore_axis_name="s", num_cores=NC, num_subcores=NS
)


def _deg_body(cidx_hbm, ones_hbm, z16_hbm, out_hbm, ones_v, cslab_v, acc_sh, sem):
    cid = lax.axis_index("c")
    sid = lax.axis_index("s")
    wid = cid * NS + sid

    pltpu.sync_copy(ones_hbm, ones_v)
    pltpu.sync_copy(
        z16_hbm.at[pl.ds(sid * RPT, RPT)], acc_sh.at[pl.ds(sid * RPT, RPT)]
    )
    pltpu.sync_copy(cidx_hbm.at[wid], cslab_v)
    plsc.subcore_barrier()

    @pl.loop(0, NCHUNK // 8, step=1)
    def _(j):
        pltpu.async_copy(ones_v, acc_sh.at[cslab_v.at[j]], sem, add=True)

    @pl.loop(0, NCHUNK // 8, step=1)
    def _(j):
        pltpu.make_async_copy(ones_v, acc_sh.at[cslab_v.at[0]], sem).wait()

    plsc.subcore_barrier()
    pltpu.sync_copy(
        acc_sh.at[pl.ds(sid * RPT, RPT)], out_hbm.at[cid, pl.ds(sid * RPT, RPT)]
    )


@functools.partial(
    pl.kernel,
    out_type=jax.ShapeDtypeStruct((NC, NPAD, 16), jnp.float32),
    mesh=_sc_mesh,
    compiler_params=pltpu.CompilerParams(use_tc_tiling_on_sc=False),
    scratch_types=[
        pltpu.VMEM((8 * CH, 16), jnp.float32),
        pltpu.VMEM((NCHUNK // 8, 8 * CH), jnp.int32),
        pltpu.VMEM_SHARED((NPAD, 16), jnp.float32),
        pltpu.SemaphoreType.DMA,
    ],
)
def _deg_sc(cidx_hbm, ones_hbm, z16_hbm, out_hbm, ones_v, cslab_v, acc_sh, sem):
    _deg_body(cidx_hbm, ones_hbm, z16_hbm, out_hbm, ones_v, cslab_v, acc_sh, sem)


def _scat_body(y_hbm, ridx_hbm, cidx_hbm, z_hbm, out_hbm,
               rslab_v, cslab_v, bufs, acc_sh, y_sh, gsems, ssems):
    cid = lax.axis_index("c")
    sid = lax.axis_index("s")
    wid = cid * NS + sid

    def fire_g(i, k):
        pltpu.async_copy(y_sh.at[rslab_v.at[i]], bufs[k], gsems[k])

    def drain_g(k):
        pltpu.make_async_copy(y_sh.at[rslab_v.at[0]], bufs[k], gsems[k]).wait()

    def fire_s(i, k):
        pltpu.async_copy(bufs[k], acc_sh.at[cslab_v.at[i]], ssems[k], add=True)

    def drain_s(k):
        pltpu.make_async_copy(bufs[k], acc_sh.at[cslab_v.at[0]], ssems[k]).wait()

    pltpu.sync_copy(
        z_hbm.at[pl.ds(sid * RPT, RPT)], acc_sh.at[pl.ds(sid * RPT, RPT)]
    )
    pltpu.sync_copy(
        y_hbm.at[pl.ds(sid * RPT, RPT)], y_sh.at[pl.ds(sid * RPT, RPT)]
    )
    pltpu.sync_copy(ridx_hbm.at[wid], rslab_v)
    pltpu.sync_copy(cidx_hbm.at[wid], cslab_v)
    plsc.subcore_barrier()

    fire_g(0, 0)

    @pl.loop(0, NSUP, step=2)
    def _(i):
        drain_g(0)
        fire_s(i, 0)
        fire_g(i + 1, 1)
        drain_s(0)
        drain_g(1)
        fire_s(i + 1, 1)

        @pl.when(i + 2 < NSUP)
        def _():
            fire_g(i + 2, 0)

        drain_s(1)

    plsc.subcore_barrier()
    pltpu.sync_copy(
        acc_sh.at[pl.ds(sid * RPT, RPT)], out_hbm.at[cid, pl.ds(sid * RPT, RPT)]
    )


@functools.partial(
    pl.kernel,
    out_type=jax.ShapeDtypeStruct((NC, NPAD, W), jnp.float32),
    mesh=_sc_mesh,
    compiler_params=pltpu.CompilerParams(use_tc_tiling_on_sc=False),
    scratch_types=(
        [pltpu.VMEM((NSUP, SCH), jnp.int32)] * 2
        + [pltpu.VMEM((SCH, W), jnp.float32)] * 2
        + [pltpu.VMEM_SHARED((NPAD, W), jnp.float32)] * 2
        + [pltpu.SemaphoreType.DMA] * 4
    ),
)
def _scat_sc(y_hbm, ridx_hbm, cidx_hbm, z_hbm, out_hbm,
             rslab_v, cslab_v, buf0, buf1, acc_sh, y_sh, g0, g1, s0, s1):
    _scat_body(y_hbm, ridx_hbm, cidx_hbm, z_hbm, out_hbm,
               rslab_v, cslab_v, (buf0, buf1), acc_sh, y_sh, (g0, g1), (s0, s1))


def _mlp_body(xp_ref, w0_ref, b0_ref, w1_ref, b1_ref, h_ref):
    g = jnp.dot(xp_ref[...], w0_ref[...], preferred_element_type=jnp.float32)
    g = jnp.maximum(g + b0_ref[...], 0.0)
    h_ref[...] = (
        jnp.dot(g, w1_ref[...], preferred_element_type=jnp.float32) + b1_ref[...]
    )


def _mlp_tc(xp, w0, b0r, w1p, b1p):
    return pl.pallas_call(
        _mlp_body,
        out_shape=jax.ShapeDtypeStruct((NPAD, W), jnp.float32),
    )(xp, w0, b0r, w1p, b1p)


def _prep_body(h_ref, degp_ref, dinv_ref, y0_ref):
    deg = degp_ref[0, :, 0:1] + degp_ref[1, :, 0:1] + 1.0
    dinv = jnp.broadcast_to(lax.rsqrt(deg), (NPAD, W))
    dinv_ref[...] = dinv
    y0_ref[...] = dinv * h_ref[...]


def _prep_tc(h, degp):
    return pl.pallas_call(
        _prep_body,
        out_shape=(
            jax.ShapeDtypeStruct((NPAD, W), jnp.float32),
            jax.ShapeDtypeStruct((NPAD, W), jnp.float32),
        ),
    )(h, degp)


def _combine_body(sp_ref, x_ref, h_ref, dinv_ref, xn_ref, yn_ref):
    dinv = dinv_ref[...]
    s = sp_ref[0] + sp_ref[1]
    xn = (1.0 - ALPHA) * (dinv * s + dinv * dinv * x_ref[...]) + ALPHA * h_ref[...]
    xn_ref[...] = xn
    yn_ref[...] = dinv * xn


def _combine_tc(sp, x, h, dinv):
    return pl.pallas_call(
        _combine_body,
        out_shape=(
            jax.ShapeDtypeStruct((NPAD, W), jnp.float32),
            jax.ShapeDtypeStruct((NPAD, W), jnp.float32),
        ),
    )(sp, x, h, dinv)


def _last_body(sp_ref, x_ref, h_ref, dinv_ref, out_ref):
    dinv = dinv_ref[...]
    s = sp_ref[0] + sp_ref[1]
    xn = (1.0 - ALPHA) * (dinv * s + dinv * dinv * x_ref[...]) + ALPHA * h_ref[...]
    mask = lax.broadcasted_iota(jnp.int32, (NPAD, W), 1) < C
    xm = jnp.where(mask, xn, -jnp.inf)
    m = jnp.max(xm, axis=1, keepdims=True)
    e = jnp.where(mask, jnp.exp(xm - m), 0.0)
    lse = jnp.log(jnp.sum(e, axis=1, keepdims=True)) + m
    out_ref[...] = xn - lse


def _last_tc(sp, x, h, dinv):
    return pl.pallas_call(
        _last_body,
        out_shape=jax.ShapeDtypeStruct((NPAD, W), jnp.float32),
    )(sp, x, h, dinv)


def kernel(inputs, edge_index, W0, b0, W1, b1):
    row = edge_index[0].astype(jnp.int32)
    col = edge_index[1].astype(jnp.int32)
    npad_e = EPAD - E
    ridx = jnp.concatenate([row, jnp.zeros((npad_e,), jnp.int32)])
    cidx = jnp.concatenate([col, jnp.full((npad_e,), DUMMY, jnp.int32)])
    ridx = ridx.reshape(NWORK, NSUP, SCH)
    cidx = cidx.reshape(NWORK, NSUP, SCH)
    cidx8 = cidx.reshape(NWORK, NCHUNK // 8, 8 * CH)

    xp = jnp.zeros((NPAD, D), jnp.float32).at[:N].set(inputs)
    w1p = jnp.zeros((H, W), jnp.float32).at[:, :C].set(W1)
    b1p = jnp.zeros((1, W), jnp.float32).at[0, :C].set(b1)
    b0r = b0.reshape(1, H)
    ones16 = jnp.ones((8 * CH, 16), jnp.float32)
    z16 = jnp.zeros((NPAD, 16), jnp.float32)
    z48 = jnp.zeros((NPAD, W), jnp.float32)

    degp = _deg_sc(cidx8, ones16, z16)
    h = _mlp_tc(xp, W0, b0r, w1p, b1p)
    dinv, y = _prep_tc(h, degp)
    x = h
    for _ in range(K - 1):
        sp = _scat_sc(y, ridx, cidx, z48)
        x, y = _combine_tc(sp, x, h, dinv)
    sp = _scat_sc(y, ridx, cidx, z48)
    out = _last_tc(sp, x, h, dinv)
    return out[:N, :C]


# speedup vs baseline: 3.0847x; 1.2928x over previous
import functools

import jax
import jax.numpy as jnp
from jax import lax
from jax.experimental import pallas as pl
from jax.experimental.pallas import tpu as pltpu
from jax.experimental.pallas import tpu_sc as plsc

N = 10000
E = 320000
D = 128
H = 64
C = 40
K = 10
ALPHA = 0.1

W = 48
NPAD = 10112
DUMMY = N
NC, NS = 2, 16
NWORK = NC * NS
CH = 128
NCHUNK = 80
EPT = CH * NCHUNK
EPAD = EPT * NWORK
RPT = NPAD // NS
FLAT = NPAD * W // 128

SR = 4
SCH = SR * CH
NSUP = EPT // SCH

_sc_mesh = plsc.VectorSubcoreMesh(
    core_axis_name="c", subcore_axis_name="s", num_cores=NC, num_subcores=NS
)


def _deg_body(cidx_hbm, ones_hbm, z16_hbm, out_hbm, ones_v, cslab_v, acc_sh, sem):
    cid = lax.axis_index("c")
    sid = lax.axis_index("s")
    wid = cid * NS + sid

    pltpu.sync_copy(ones_hbm, ones_v)
    pltpu.sync_copy(
        z16_hbm.at[pl.ds(sid * RPT, RPT)], acc_sh.at[pl.ds(sid * RPT, RPT)]
    )
    pltpu.sync_copy(cidx_hbm.at[wid], cslab_v)
    plsc.subcore_barrier()

    @pl.loop(0, NCHUNK // 8, step=1)
    def _(j):
        pltpu.async_copy(ones_v, acc_sh.at[cslab_v.at[j]], sem, add=True)

    @pl.loop(0, NCHUNK // 8, step=1)
    def _(j):
        pltpu.make_async_copy(ones_v, acc_sh.at[cslab_v.at[0]], sem).wait()

    plsc.subcore_barrier()
    pltpu.sync_copy(
        acc_sh.at[pl.ds(sid * RPT, RPT)], out_hbm.at[cid, pl.ds(sid * RPT, RPT)]
    )


@functools.partial(
    pl.kernel,
    out_type=jax.ShapeDtypeStruct((NC, NPAD, 16), jnp.float32),
    mesh=_sc_mesh,
    compiler_params=pltpu.CompilerParams(use_tc_tiling_on_sc=False),
    scratch_types=[
        pltpu.VMEM((8 * CH, 16), jnp.float32),
        pltpu.VMEM((NCHUNK // 8, 8 * CH), jnp.int32),
        pltpu.VMEM_SHARED((NPAD, 16), jnp.float32),
        pltpu.SemaphoreType.DMA,
    ],
)
def _deg_sc(cidx_hbm, ones_hbm, z16_hbm, out_hbm, ones_v, cslab_v, acc_sh, sem):
    _deg_body(cidx_hbm, ones_hbm, z16_hbm, out_hbm, ones_v, cslab_v, acc_sh, sem)


def _scat_body(y_hbm, ridx_hbm, cidx_hbm, z_hbm, out_hbm,
               rslab_v, cslab_v, bufs, acc_sh, y_sh, gsems, ssems):
    cid = lax.axis_index("c")
    sid = lax.axis_index("s")
    wid = cid * NS + sid

    def fire_g(i, k):
        pltpu.async_copy(y_sh.at[rslab_v.at[i]], bufs[k], gsems[k])

    def drain_g(k):
        pltpu.make_async_copy(y_sh.at[rslab_v.at[0]], bufs[k], gsems[k]).wait()

    def fire_s(i, k):
        pltpu.async_copy(bufs[k], acc_sh.at[cslab_v.at[i]], ssems[k], add=True)

    def drain_s(k):
        pltpu.make_async_copy(bufs[k], acc_sh.at[cslab_v.at[0]], ssems[k]).wait()

    @pl.when(cid == 0)
    def _():
        pltpu.sync_copy(
            y_hbm.at[pl.ds(sid * RPT, RPT)], acc_sh.at[pl.ds(sid * RPT, RPT)]
        )

    @pl.when(cid != 0)
    def _():
        pltpu.sync_copy(
            z_hbm.at[pl.ds(sid * RPT, RPT)], acc_sh.at[pl.ds(sid * RPT, RPT)]
        )

    pltpu.sync_copy(
        y_hbm.at[pl.ds(sid * RPT, RPT)], y_sh.at[pl.ds(sid * RPT, RPT)]
    )
    pltpu.sync_copy(ridx_hbm.at[wid], rslab_v)
    pltpu.sync_copy(cidx_hbm.at[wid], cslab_v)
    plsc.subcore_barrier()

    fire_g(0, 0)

    @pl.loop(0, NSUP, step=2)
    def _(i):
        drain_g(0)
        fire_s(i, 0)
        fire_g(i + 1, 1)
        drain_s(0)
        drain_g(1)
        fire_s(i + 1, 1)

        @pl.when(i + 2 < NSUP)
        def _():
            fire_g(i + 2, 0)

        drain_s(1)

    plsc.subcore_barrier()
    pltpu.sync_copy(
        acc_sh.at[pl.ds(sid * RPT, RPT)], out_hbm.at[cid, pl.ds(sid * RPT, RPT)]
    )


@functools.partial(
    pl.kernel,
    out_type=jax.ShapeDtypeStruct((NC, NPAD, W), jnp.float32),
    mesh=_sc_mesh,
    compiler_params=pltpu.CompilerParams(use_tc_tiling_on_sc=False),
    scratch_types=(
        [pltpu.VMEM((NSUP, SCH), jnp.int32)] * 2
        + [pltpu.VMEM((SCH, W), jnp.float32)] * 2
        + [pltpu.VMEM_SHARED((NPAD, W), jnp.float32)] * 2
        + [pltpu.SemaphoreType.DMA] * 4
    ),
)
def _scat_sc(y_hbm, ridx_hbm, cidx_hbm, z_hbm, out_hbm,
             rslab_v, cslab_v, buf0, buf1, acc_sh, y_sh, g0, g1, s0, s1):
    _scat_body(y_hbm, ridx_hbm, cidx_hbm, z_hbm, out_hbm,
               rslab_v, cslab_v, (buf0, buf1), acc_sh, y_sh, (g0, g1), (s0, s1))


def _mlp_body(xp_ref, w0_ref, b0_ref, w1_ref, b1_ref, h_ref):
    g = jnp.dot(xp_ref[...], w0_ref[...], preferred_element_type=jnp.float32)
    g = jnp.maximum(g + b0_ref[...], 0.0)
    h_ref[...] = (
        jnp.dot(g, w1_ref[...], preferred_element_type=jnp.float32) + b1_ref[...]
    )


def _mlp_tc(xp, w0, b0r, w1p, b1p):
    return pl.pallas_call(
        _mlp_body,
        out_shape=jax.ShapeDtypeStruct((NPAD, W), jnp.float32),
    )(xp, w0, b0r, w1p, b1p)


def _prep_body(h_ref, degp_ref, y0_ref, d2s_ref, y0s_ref, dinv_ref):
    deg = degp_ref[0, :, 0:1] + degp_ref[1, :, 0:1] + 1.0
    dinv = jnp.broadcast_to(lax.rsqrt(deg), (NPAD, W))
    y0 = dinv * h_ref[...]
    y0_ref[...] = y0
    d2s_ref[...] = (1.0 - ALPHA) * dinv * dinv
    y0s_ref[...] = ALPHA * y0
    dinv_ref[...] = dinv


def _prep_tc(h, degp):
    s = jax.ShapeDtypeStruct((NPAD, W), jnp.float32)
    return pl.pallas_call(_prep_body, out_shape=(s, s, s, s))(h, degp)


def _combine_body(spf_ref, d2sf_ref, y0sf_ref, yf_ref):
    yf_ref[...] = d2sf_ref[...] * (spf_ref[0] + spf_ref[1]) + y0sf_ref[...]


def _combine_tc(spf, d2sf, y0sf):
    return pl.pallas_call(
        _combine_body,
        out_shape=jax.ShapeDtypeStruct((FLAT, 128), jnp.float32),
    )(spf, d2sf, y0sf)


def _soft_body(y_ref, dinv_ref, out_ref):
    xn = y_ref[...] / dinv_ref[...]
    mask = lax.broadcasted_iota(jnp.int32, (NPAD, W), 1) < C
    xm = jnp.where(mask, xn, -jnp.inf)
    m = jnp.max(xm, axis=1, keepdims=True)
    e = jnp.where(mask, jnp.exp(xm - m), 0.0)
    lse = jnp.log(jnp.sum(e, axis=1, keepdims=True)) + m
    out_ref[...] = xn - lse


def _soft_tc(y, dinvb):
    return pl.pallas_call(
        _soft_body,
        out_shape=jax.ShapeDtypeStruct((NPAD, W), jnp.float32),
    )(y, dinvb)


def kernel(inputs, edge_index, W0, b0, W1, b1):
    row = edge_index[0].astype(jnp.int32)
    col = edge_index[1].astype(jnp.int32)
    npad_e = EPAD - E
    ridx = jnp.concatenate([row, jnp.zeros((npad_e,), jnp.int32)])
    cidx = jnp.concatenate([col, jnp.full((npad_e,), DUMMY, jnp.int32)])
    ridx = ridx.reshape(NWORK, NSUP, SCH)
    cidx = cidx.reshape(NWORK, NSUP, SCH)
    cidx8 = cidx.reshape(NWORK, NCHUNK // 8, 8 * CH)

    xp = jnp.zeros((NPAD, D), jnp.float32).at[:N].set(inputs)
    w1p = jnp.zeros((H, W), jnp.float32).at[:, :C].set(W1)
    b1p = jnp.zeros((1, W), jnp.float32).at[0, :C].set(b1)
    b0r = b0.reshape(1, H)
    ones16 = jnp.ones((8 * CH, 16), jnp.float32)
    z16 = jnp.zeros((NPAD, 16), jnp.float32)
    z48 = jnp.zeros((NPAD, W), jnp.float32)

    degp = _deg_sc(cidx8, ones16, z16)
    h = _mlp_tc(xp, W0, b0r, w1p, b1p)
    y0, d2s, y0s, dinvb = _prep_tc(h, degp)
    d2sf = d2s.reshape(FLAT, 128)
    y0sf = y0s.reshape(FLAT, 128)
    y = y0
    for _ in range(K):
        sp = _scat_sc(y, ridx, cidx, z48)
        yf = _combine_tc(sp.reshape(NC, FLAT, 128), d2sf, y0sf)
        y = yf.reshape(NPAD, W)
    out = _soft_tc(y, dinvb)
    return out[:N, :C]


# speedup vs baseline: 3.1147x; 1.0097x over previous
import functools

import jax
import jax.numpy as jnp
from jax import lax
from jax.experimental import pallas as pl
from jax.experimental.pallas import tpu as pltpu
from jax.experimental.pallas import tpu_sc as plsc

N = 10000
E = 320000
D = 128
H = 64
C = 40
K = 10
ALPHA = 0.1

W = 48
NPAD = 10112
DUMMY = N
NC, NS = 2, 16
NWORK = NC * NS
CH = 128
NCHUNK = 80
EPT = CH * NCHUNK
EPAD = EPT * NWORK
RPT = NPAD // NS
FLAT = NPAD * W // 128

SR = 4
SCH = SR * CH
NSUP = EPT // SCH

_sc_mesh = plsc.VectorSubcoreMesh(
    core_axis_name="c", subcore_axis_name="s", num_cores=NC, num_subcores=NS
)


def _deg_body(cidx_hbm, ones_hbm, z16_hbm, out_hbm, ones_v, cslab_v, acc_sh, sem):
    cid = lax.axis_index("c")
    sid = lax.axis_index("s")
    wid = cid * NS + sid

    pltpu.sync_copy(ones_hbm, ones_v)
    pltpu.sync_copy(
        z16_hbm.at[pl.ds(sid * RPT, RPT)], acc_sh.at[pl.ds(sid * RPT, RPT)]
    )
    pltpu.sync_copy(cidx_hbm.at[wid], cslab_v)
    plsc.subcore_barrier()

    @pl.loop(0, NCHUNK // 8, step=1)
    def _(j):
        pltpu.async_copy(ones_v, acc_sh.at[cslab_v.at[j]], sem, add=True)

    @pl.loop(0, NCHUNK // 8, step=1)
    def _(j):
        pltpu.make_async_copy(ones_v, acc_sh.at[cslab_v.at[0]], sem).wait()

    plsc.subcore_barrier()
    pltpu.sync_copy(
        acc_sh.at[pl.ds(sid * RPT, RPT)], out_hbm.at[cid, pl.ds(sid * RPT, RPT)]
    )


@functools.partial(
    pl.kernel,
    out_type=jax.ShapeDtypeStruct((NC, NPAD, 16), jnp.float32),
    mesh=_sc_mesh,
    compiler_params=pltpu.CompilerParams(use_tc_tiling_on_sc=False),
    scratch_types=[
        pltpu.VMEM((8 * CH, 16), jnp.float32),
        pltpu.VMEM((NCHUNK // 8, 8 * CH), jnp.int32),
        pltpu.VMEM_SHARED((NPAD, 16), jnp.float32),
        pltpu.SemaphoreType.DMA,
    ],
)
def _deg_sc(cidx_hbm, ones_hbm, z16_hbm, out_hbm, ones_v, cslab_v, acc_sh, sem):
    _deg_body(cidx_hbm, ones_hbm, z16_hbm, out_hbm, ones_v, cslab_v, acc_sh, sem)


def _scat_body(y_hbm, ridx_hbm, cidx_hbm, z_hbm, out_hbm,
               rslab_v, cslab_v, bufs, acc_sh, y_sh, gsems, ssems):
    cid = lax.axis_index("c")
    sid = lax.axis_index("s")
    wid = cid * NS + sid

    NSUB = 4
    SUB = SCH // NSUB

    def fire_g(i, k):
        for s in range(NSUB):
            pltpu.async_copy(
                y_sh.at[rslab_v.at[i, pl.ds(s * SUB, SUB)]],
                bufs[k].at[pl.ds(s * SUB, SUB)],
                gsems[k],
            )

    def drain_g(k):
        for s in range(NSUB):
            pltpu.make_async_copy(
                y_sh.at[rslab_v.at[0, pl.ds(0, SUB)]],
                bufs[k].at[pl.ds(s * SUB, SUB)],
                gsems[k],
            ).wait()

    def fire_s(i, k):
        pltpu.async_copy(bufs[k], acc_sh.at[cslab_v.at[i]], ssems[k], add=True)

    def drain_s(k):
        pltpu.make_async_copy(bufs[k], acc_sh.at[cslab_v.at[0]], ssems[k]).wait()

    @pl.when(cid == 0)
    def _():
        pltpu.sync_copy(
            y_hbm.at[pl.ds(sid * RPT, RPT)], acc_sh.at[pl.ds(sid * RPT, RPT)]
        )

    @pl.when(cid != 0)
    def _():
        pltpu.sync_copy(
            z_hbm.at[pl.ds(sid * RPT, RPT)], acc_sh.at[pl.ds(sid * RPT, RPT)]
        )

    pltpu.sync_copy(
        y_hbm.at[pl.ds(sid * RPT, RPT)], y_sh.at[pl.ds(sid * RPT, RPT)]
    )
    pltpu.sync_copy(ridx_hbm.at[wid], rslab_v)
    pltpu.sync_copy(cidx_hbm.at[wid], cslab_v)
    plsc.subcore_barrier()

    fire_g(0, 0)

    @pl.loop(0, NSUP, step=2)
    def _(i):
        drain_g(0)
        fire_s(i, 0)
        fire_g(i + 1, 1)
        drain_s(0)
        drain_g(1)
        fire_s(i + 1, 1)

        @pl.when(i + 2 < NSUP)
        def _():
            fire_g(i + 2, 0)

        drain_s(1)

    plsc.subcore_barrier()
    pltpu.sync_copy(
        acc_sh.at[pl.ds(sid * RPT, RPT)], out_hbm.at[cid, pl.ds(sid * RPT, RPT)]
    )


@functools.partial(
    pl.kernel,
    out_type=jax.ShapeDtypeStruct((NC, NPAD, W), jnp.float32),
    mesh=_sc_mesh,
    compiler_params=pltpu.CompilerParams(use_tc_tiling_on_sc=False),
    scratch_types=(
        [pltpu.VMEM((NSUP, SCH), jnp.int32)] * 2
        + [pltpu.VMEM((SCH, W), jnp.float32)] * 2
        + [pltpu.VMEM_SHARED((NPAD, W), jnp.float32)] * 2
        + [pltpu.SemaphoreType.DMA] * 4
    ),
)
def _scat_sc(y_hbm, ridx_hbm, cidx_hbm, z_hbm, out_hbm,
             rslab_v, cslab_v, buf0, buf1, acc_sh, y_sh, g0, g1, s0, s1):
    _scat_body(y_hbm, ridx_hbm, cidx_hbm, z_hbm, out_hbm,
               rslab_v, cslab_v, (buf0, buf1), acc_sh, y_sh, (g0, g1), (s0, s1))


def _mlp_body(xp_ref, w0_ref, b0_ref, w1_ref, b1_ref, h_ref):
    g = jnp.dot(xp_ref[...], w0_ref[...], preferred_element_type=jnp.float32)
    g = jnp.maximum(g + b0_ref[...], 0.0)
    h_ref[...] = (
        jnp.dot(g, w1_ref[...], preferred_element_type=jnp.float32) + b1_ref[...]
    )


def _mlp_tc(xp, w0, b0r, w1p, b1p):
    return pl.pallas_call(
        _mlp_body,
        out_shape=jax.ShapeDtypeStruct((NPAD, W), jnp.float32),
    )(xp, w0, b0r, w1p, b1p)


def _prep_body(h_ref, degp_ref, y0_ref, d2s_ref, y0s_ref, dinv_ref):
    deg = degp_ref[0, :, 0:1] + degp_ref[1, :, 0:1] + 1.0
    dinv = jnp.broadcast_to(lax.rsqrt(deg), (NPAD, W))
    y0 = dinv * h_ref[...]
    y0_ref[...] = y0
    d2s_ref[...] = (1.0 - ALPHA) * dinv * dinv
    y0s_ref[...] = ALPHA * y0
    dinv_ref[...] = dinv


def _prep_tc(h, degp):
    s = jax.ShapeDtypeStruct((NPAD, W), jnp.float32)
    return pl.pallas_call(_prep_body, out_shape=(s, s, s, s))(h, degp)


def _combine_body(spf_ref, d2sf_ref, y0sf_ref, yf_ref):
    yf_ref[...] = d2sf_ref[...] * (spf_ref[0] + spf_ref[1]) + y0sf_ref[...]


def _combine_tc(spf, d2sf, y0sf):
    return pl.pallas_call(
        _combine_body,
        out_shape=jax.ShapeDtypeStruct((FLAT, 128), jnp.float32),
    )(spf, d2sf, y0sf)


def _soft_body(y_ref, dinv_ref, out_ref):
    xn = y_ref[...] / dinv_ref[...]
    mask = lax.broadcasted_iota(jnp.int32, (NPAD, W), 1) < C
    xm = jnp.where(mask, xn, -jnp.inf)
    m = jnp.max(xm, axis=1, keepdims=True)
    e = jnp.where(mask, jnp.exp(xm - m), 0.0)
    lse = jnp.log(jnp.sum(e, axis=1, keepdims=True)) + m
    out_ref[...] = xn - lse


def _soft_tc(y, dinvb):
    return pl.pallas_call(
        _soft_body,
        out_shape=jax.ShapeDtypeStruct((NPAD, W), jnp.float32),
    )(y, dinvb)


def kernel(inputs, edge_index, W0, b0, W1, b1):
    row = edge_index[0].astype(jnp.int32)
    col = edge_index[1].astype(jnp.int32)
    npad_e = EPAD - E
    ridx = jnp.concatenate([row, jnp.zeros((npad_e,), jnp.int32)])
    cidx = jnp.concatenate([col, jnp.full((npad_e,), DUMMY, jnp.int32)])
    ridx = ridx.reshape(NWORK, NSUP, SCH)
    cidx = cidx.reshape(NWORK, NSUP, SCH)
    cidx8 = cidx.reshape(NWORK, NCHUNK // 8, 8 * CH)

    xp = jnp.zeros((NPAD, D), jnp.float32).at[:N].set(inputs)
    w1p = jnp.zeros((H, W), jnp.float32).at[:, :C].set(W1)
    b1p = jnp.zeros((1, W), jnp.float32).at[0, :C].set(b1)
    b0r = b0.reshape(1, H)
    ones16 = jnp.ones((8 * CH, 16), jnp.float32)
    z16 = jnp.zeros((NPAD, 16), jnp.float32)
    z48 = jnp.zeros((NPAD, W), jnp.float32)

    degp = _deg_sc(cidx8, ones16, z16)
    h = _mlp_tc(xp, W0, b0r, w1p, b1p)
    y0, d2s, y0s, dinvb = _prep_tc(h, degp)
    d2sf = d2s.reshape(FLAT, 128)
    y0sf = y0s.reshape(FLAT, 128)
    y = y0
    for _ in range(K):
        sp = _scat_sc(y, ridx, cidx, z48)
        yf = _combine_tc(sp.reshape(NC, FLAT, 128), d2sf, y0sf)
        y = yf.reshape(NPAD, W)
    out = _soft_tc(y, dinvb)
    return out[:N, :C]
